# baseline (speedup 1.0000x reference)
import jax
import jax.numpy as jnp
from jax import lax
from jax.experimental import pallas as pl
from jax.experimental.pallas import tpu as pltpu, tpu_sc as plsc

B = 16384
DC = 13
SC_ = 26
NC, NS, L = 2, 16, 16
NW = NC * NS
BW = B // NW
NCH = BW // L


def _sc_body(sparse_hbm, table_hbm, gsum_hbm, idx_v, gv_v, gs_v, sem):
    wid = lax.axis_index("s") * NC + lax.axis_index("c")
    base = wid * BW
    iota16 = lax.iota(jnp.int32, L)

    def stage_idx(idx2_v):
        pltpu.sync_copy(sparse_hbm.at[pl.ds(base, BW), :], idx2_v)

        def reformat(c, _):
            rows = iota16 + c * L
            for j in range(SC_):
                idx_v[pl.ds(j * BW + c * L, L)] = plsc.load_gather(
                    idx2_v, [rows, jnp.full((L,), j, jnp.int32)])
            return _

        lax.fori_loop(0, NCH, reformat, 0)

    pl.run_scoped(stage_idx, pltpu.VMEM((BW, SC_), jnp.int32))

    pltpu.async_copy(table_hbm.at[idx_v], gv_v, sem).wait()

    def chunk(c, _):
        acc = gv_v[pl.ds(c * L, L)]
        for j in range(1, SC_):
            acc = acc + gv_v[pl.ds(j * BW + c * L, L)]
        gs_v[pl.ds(c * L, L)] = acc
        return _

    lax.fori_loop(0, NCH, chunk, 0)
    pltpu.sync_copy(gs_v, gsum_hbm.at[pl.ds(base, BW)])


def _tc_body(dense_ref, w_ref, gsum_ref, bias_ref, out_ref):
    d = dense_ref[...]
    w = w_ref[...]
    x = jax.lax.dot_general(d, w, (((1,), (0,)), ((), ())),
                            preferred_element_type=jnp.float32)
    out_ref[...] = 1.0 / (1.0 + jnp.exp(-(x + gsum_ref[...] + bias_ref[0, 0])))


@jax.jit
def _run(dense_input, sparse_input, w_dense, table, bias):
    mesh = plsc.VectorSubcoreMesh(core_axis_name="c", subcore_axis_name="s")
    sc_k = pl.kernel(
        _sc_body,
        out_type=jax.ShapeDtypeStruct((B,), jnp.float32),
        mesh=mesh,
        compiler_params=pltpu.CompilerParams(needs_layout_passes=False),
        scratch_types=[
            pltpu.VMEM((BW * SC_,), jnp.int32),
            pltpu.VMEM((BW * SC_,), jnp.float32),
            pltpu.VMEM((BW,), jnp.float32),
            pltpu.SemaphoreType.DMA,
        ],
    )
    gsum = sc_k(sparse_input, table)
    return pl.pallas_call(
        _tc_body,
        out_shape=jax.ShapeDtypeStruct((B, 1), jnp.float32),
    )(dense_input, w_dense, gsum.reshape(B, 1), bias.reshape(1, 1))


def kernel(dense_input, sparse_input, w_dense, w_sparse, bias):
    return _run(dense_input, sparse_input, w_dense, w_sparse[:, 0], bias)


# speedup vs baseline: 1.0662x; 1.0662x over previous
import jax
import jax.numpy as jnp
from jax import lax
from jax.experimental import pallas as pl
from jax.experimental.pallas import tpu as pltpu, tpu_sc as plsc

B = 16384
DC = 13
SC_ = 26
NC, NS, L = 2, 16, 16
NW = NC * NS
BW = B // NW
NCH = BW // L


def _sc_body(sparse_hbm, table_hbm, gsum_hbm, idx_v, gv_v, gs_v, sem):
    wid = lax.axis_index("s") * NC + lax.axis_index("c")
    base = wid * BW
    iota16 = lax.iota(jnp.int32, L)

    def stage_idx(idx2_v):
        pltpu.sync_copy(sparse_hbm.at[pl.ds(base, BW), :], idx2_v)

        def reformat(c, _):
            rows = iota16 + c * L
            for j in range(SC_):
                idx_v[pl.ds(j * BW + c * L, L)] = plsc.load_gather(
                    idx2_v, [rows, jnp.full((L,), j, jnp.int32)])
            return _

        lax.fori_loop(0, NCH, reformat, 0)

    pl.run_scoped(stage_idx, pltpu.VMEM((BW, SC_), jnp.int32))

    pltpu.async_copy(table_hbm.at[idx_v], gv_v, sem).wait()

    def chunk(c, _):
        acc = gv_v[pl.ds(c * L, L)]
        for j in range(1, SC_):
            acc = acc + gv_v[pl.ds(j * BW + c * L, L)]
        gs_v[pl.ds(c * L, L)] = acc
        return _

    lax.fori_loop(0, NCH, chunk, 0)
    pltpu.sync_copy(gs_v, gsum_hbm.at[pl.ds(base, BW)])


def _tc_body(dense_ref, w_ref, gsum_ref, bias_ref, out_ref):
    d = dense_ref[...]
    w = w_ref[...]
    x = jax.lax.dot_general(d, w, (((1,), (0,)), ((), ())),
                            preferred_element_type=jnp.float32)
    g = gsum_ref[...].reshape(B, 1)
    out_ref[...] = 1.0 / (1.0 + jnp.exp(-(x + g + bias_ref[0, 0])))


@jax.jit
def _run(dense_input, sparse_input, w_dense, table, bias):
    mesh = plsc.VectorSubcoreMesh(core_axis_name="c", subcore_axis_name="s")
    sc_k = pl.kernel(
        _sc_body,
        out_type=jax.ShapeDtypeStruct((B,), jnp.float32),
        mesh=mesh,
        compiler_params=pltpu.CompilerParams(needs_layout_passes=False),
        scratch_types=[
            pltpu.VMEM((BW * SC_,), jnp.int32),
            pltpu.VMEM((BW * SC_,), jnp.float32),
            pltpu.VMEM((BW,), jnp.float32),
            pltpu.SemaphoreType.DMA,
        ],
    )
    gsum = sc_k(sparse_input, table)
    return pl.pallas_call(
        _tc_body,
        out_shape=jax.ShapeDtypeStruct((B, 1), jnp.float32),
    )(dense_input, w_dense, gsum, bias.reshape(1, 1))


def kernel(dense_input, sparse_input, w_dense, w_sparse, bias):
    return _run(dense_input, sparse_input, w_dense, w_sparse[:, 0], bias)


# speedup vs baseline: 1.0669x; 1.0006x over previous
import jax
import jax.numpy as jnp
from jax import lax
from jax.experimental import pallas as pl
from jax.experimental.pallas import tpu as pltpu, tpu_sc as plsc

B = 16384
DC = 13
SC_ = 26
NC, NS, L = 2, 16, 16
NW = NC * NS
BW = B // NW
NCH = BW // L


def _sc_body(sparse_hbm, table_hbm, gsum_hbm, idx_v, gv_v, gs_v, sem):
    wid = lax.axis_index("s") * NC + lax.axis_index("c")
    base = wid * BW
    iota16 = lax.iota(jnp.int32, L)

    def stage_idx(idx2_v):
        pltpu.sync_copy(sparse_hbm.at[pl.ds(base, BW), :], idx2_v)

        def reformat(c, _):
            rows = iota16 + c * L
            for j in range(SC_):
                idx_v[pl.ds(j * BW + c * L, L)] = plsc.load_gather(
                    idx2_v, [rows, jnp.full((L,), j, jnp.int32)])
            return _

        lax.fori_loop(0, NCH, reformat, 0)

    pl.run_scoped(stage_idx, pltpu.VMEM((BW, SC_), jnp.int32))

    pltpu.async_copy(table_hbm.at[idx_v], gv_v, sem).wait()

    def chunk(c, _):
        acc = gv_v[pl.ds(c * L, L)]
        for j in range(1, SC_):
            acc = acc + gv_v[pl.ds(j * BW + c * L, L)]
        gs_v[pl.ds(c * L, L)] = acc
        return _

    lax.fori_loop(0, NCH, chunk, 0)
    pltpu.sync_copy(gs_v, gsum_hbm.at[pl.ds(base, BW)])


def _mv_body(dense_ref, w_ref, x_ref):
    x_ref[...] = jax.lax.dot_general(
        dense_ref[...], w_ref[...], (((1,), (0,)), ((), ())),
        preferred_element_type=jnp.float32)


def _comb_body(x_ref, gsum_ref, bias_ref, out_ref):
    g = gsum_ref[...].reshape(B, 1)
    out_ref[...] = 1.0 / (1.0 + jnp.exp(-(x_ref[...] + g + bias_ref[0, 0])))


@jax.jit
def _run(dense_input, sparse_input, w_dense, table, bias):
    mesh = plsc.VectorSubcoreMesh(core_axis_name="c", subcore_axis_name="s")
    sc_k = pl.kernel(
        _sc_body,
        out_type=jax.ShapeDtypeStruct((B,), jnp.float32),
        mesh=mesh,
        compiler_params=pltpu.CompilerParams(needs_layout_passes=False),
        scratch_types=[
            pltpu.VMEM((BW * SC_,), jnp.int32),
            pltpu.VMEM((BW * SC_,), jnp.float32),
            pltpu.VMEM((BW,), jnp.float32),
            pltpu.SemaphoreType.DMA,
        ],
    )
    x = pl.pallas_call(
        _mv_body,
        out_shape=jax.ShapeDtypeStruct((B, 1), jnp.float32),
    )(dense_input, w_dense)
    gsum = sc_k(sparse_input, table)
    return pl.pallas_call(
        _comb_body,
        out_shape=jax.ShapeDtypeStruct((B, 1), jnp.float32),
    )(x, gsum, bias.reshape(1, 1))


def kernel(dense_input, sparse_input, w_dense, w_sparse, bias):
    return _run(dense_input, sparse_input, w_dense, w_sparse[:, 0], bias)


# speedup vs baseline: 1.0811x; 1.0133x over previous
import jax
import jax.numpy as jnp
from jax import lax
from jax.experimental import pallas as pl
from jax.experimental.pallas import tpu as pltpu, tpu_sc as plsc

B = 16384
DC = 13
SC_ = 26
NC, NS, L = 2, 16, 16
NW = NC * NS
BW = B // NW
NCH = BW // L


def _body(sparse_hbm, dense_hbm, wb_hbm, bias_hbm, table_hbm, out_hbm,
          idx_v, gv_v, dv_v, wb_v, bias_v, out_v, sem):
    wid = lax.axis_index("s") * NC + lax.axis_index("c")
    base = wid * BW
    pltpu.sync_copy(sparse_hbm.at[pl.ds(base * SC_, BW * SC_)], idx_v)
    pltpu.sync_copy(dense_hbm.at[pl.ds(base * DC, BW * DC)], dv_v)
    pltpu.sync_copy(wb_hbm, wb_v)
    pltpu.sync_copy(bias_hbm, bias_v)
    pltpu.async_copy(table_hbm.at[idx_v], gv_v, sem).wait()

    iota26 = lax.iota(jnp.int32, L) * SC_
    iota13 = lax.iota(jnp.int32, L) * DC

    def chunk(c, _):
        goff = c * (L * SC_)
        doff = c * (L * DC)
        acc = bias_v[...]
        for j in range(SC_):
            acc = acc + plsc.load_gather(gv_v, [iota26 + (goff + j)])
        for k in range(DC):
            acc = acc + wb_v[k] * plsc.load_gather(dv_v, [iota13 + (doff + k)])
        out_v[pl.ds(c * L, L)] = 1.0 / (1.0 + jnp.exp(-acc))
        return _

    lax.fori_loop(0, NCH, chunk, 0)
    pltpu.sync_copy(out_v, out_hbm.at[pl.ds(base, BW)])


@jax.jit
def _run(sparse_flat, dense_flat, wb, bias_b, table):
    mesh = plsc.VectorSubcoreMesh(core_axis_name="c", subcore_axis_name="s")
    k = pl.kernel(
        _body,
        out_type=jax.ShapeDtypeStruct((B,), jnp.float32),
        mesh=mesh,
        compiler_params=pltpu.CompilerParams(needs_layout_passes=False),
        scratch_types=[
            pltpu.VMEM((BW * SC_,), jnp.int32),
            pltpu.VMEM((BW * SC_,), jnp.float32),
            pltpu.VMEM((BW * DC,), jnp.float32),
            pltpu.VMEM((DC, L), jnp.float32),
            pltpu.VMEM((L,), jnp.float32),
            pltpu.VMEM((BW,), jnp.float32),
            pltpu.SemaphoreType.DMA,
        ],
    )
    return k(sparse_flat, dense_flat, wb, bias_b, table)


def kernel(dense_input, sparse_input, w_dense, w_sparse, bias):
    sparse_flat = sparse_input.reshape(-1)
    dense_flat = dense_input.reshape(-1)
    table = w_sparse.reshape(-1)
    wb = jnp.broadcast_to(w_dense.reshape(DC, 1), (DC, L))
    bias_b = jnp.broadcast_to(bias.reshape(1), (L,))
    out = _run(sparse_flat, dense_flat, wb, bias_b, table)
    return out.reshape(B, 1)
